# Initial kernel scaffold; baseline (speedup 1.0000x reference)
import jax
import jax.numpy as jnp
from jax.experimental import pallas as pl


def kernel(boxes, mount):
    raise NotImplementedError("write your pallas kernel here")


# speedup vs baseline: 4576.3557x; 4576.3557x over previous
import functools

import jax
import jax.numpy as jnp
from jax import lax
from jax.experimental import pallas as pl
from jax.experimental.pallas import tpu as pltpu
from jax.experimental.pallas import tpu_sc as plsc

W_IMG = 512
H_IMG = 512
N_BOXES = 512
MNT = 63
MNT_STRIDE = 64
N_TILES = 32
ROWS_PER_TILE = W_IMG // N_TILES
L = 16


def _render_body(boxes_hbm, mnt_hbm, out_hbm,
                 boxes_v, mnt_v, coords_v, cbuf, mbuf, canvas):
    cid = lax.axis_index("c")
    sid = lax.axis_index("s")
    wid = sid * 2 + cid
    base = wid * ROWS_PER_TILE

    pltpu.sync_copy(boxes_hbm, boxes_v)
    pltpu.sync_copy(mnt_hbm, mnt_v)

    lanes = lax.iota(jnp.int32, L)

    def coord_body(k, carry):
        fb = (k * L + lanes) * 4
        cx = plsc.load_gather(boxes_v, [fb])
        cy = plsc.load_gather(boxes_v, [fb + 1])
        bw = plsc.load_gather(boxes_v, [fb + 2])
        bh = plsc.load_gather(boxes_v, [fb + 3])
        x1 = ((cx - 0.5 * bw) * float(W_IMG)).astype(jnp.int32)
        y1 = ((cy - 0.5 * bh) * float(W_IMG)).astype(jnp.int32)
        x2 = ((cx + 0.5 * bw) * float(W_IMG)).astype(jnp.int32)
        y2 = ((cy + 0.5 * bh) * float(W_IMG)).astype(jnp.int32)
        plsc.store_scatter(coords_v, [fb], x1)
        plsc.store_scatter(coords_v, [fb + 1], y1)
        plsc.store_scatter(coords_v, [fb + 2], x2 - x1)
        plsc.store_scatter(coords_v, [fb + 3], y2 - y1)
        return carry

    lax.fori_loop(0, N_BOXES // L, coord_body, 0)

    def zero_body(k, carry):
        canvas[pl.ds(k * L, L)] = jnp.zeros((L,), jnp.float32)
        return carry

    lax.fori_loop(0, (ROWS_PER_TILE * H_IMG) // L, zero_body, 0)

    def box_body(b, carry):
        cvec = plsc.load_gather(coords_v, [b * 4 + lanes])
        x = cvec[0]
        y = cvec[1]
        w = cvec[2]
        h = cvec[3]
        r0 = jnp.maximum(x, base)
        r1 = jnp.minimum(x + w, base + ROWS_PER_TILE)

        @pl.when(r0 < r1)
        def _paint():
            wsafe = jnp.maximum(w, 1)
            hsafe = jnp.maximum(h, 1)
            nc = (h + L - 1) // L

            def pre_body(j, c2):
                col = j * L + lanes
                cc = jnp.clip((col * MNT) // hsafe, 0, MNT - 1)
                cbuf[pl.ds(j * L, L)] = cc
                mbuf[pl.ds(j * L, L)] = (col < h).astype(jnp.int32)
                return c2

            lax.fori_loop(0, nc, pre_body, 0)

            def row_body(px, c2):
                rr = jnp.clip(((px - x) * MNT) // wsafe, 0, MNT - 1)
                rbase = rr * MNT_STRIDE
                off = (px - base) * H_IMG + y

                def col_body(j, c3):
                    cc = cbuf[pl.ds(j * L, L)]
                    m = mbuf[pl.ds(j * L, L)] > 0
                    val = plsc.load_gather(mnt_v, [rbase + cc])
                    idx = off + j * L + lanes
                    cur = plsc.load_gather(canvas, [idx])
                    plsc.store_scatter(canvas, [idx],
                                       jnp.maximum(cur, val), mask=m)
                    return c3

                lax.fori_loop(0, nc, col_body, 0)
                return c2

            lax.fori_loop(r0, r1, row_body, 0)

        return carry

    lax.fori_loop(0, N_BOXES, box_body, 0)

    off = pl.multiple_of(base * H_IMG, 512)
    pltpu.sync_copy(canvas, out_hbm.at[pl.ds(off, ROWS_PER_TILE * H_IMG)])


@jax.jit
def _render(boxes_flat, mnt_flat):
    mesh = plsc.VectorSubcoreMesh(core_axis_name="c", subcore_axis_name="s")
    f = functools.partial(
        pl.kernel,
        mesh=mesh,
        compiler_params=pltpu.CompilerParams(needs_layout_passes=False),
        out_type=jax.ShapeDtypeStruct((W_IMG * H_IMG,), jnp.float32),
        scratch_types=[
            pltpu.VMEM((N_BOXES * 4,), jnp.float32),
            pltpu.VMEM((MNT_STRIDE * MNT_STRIDE,), jnp.float32),
            pltpu.VMEM((N_BOXES * 4 + L,), jnp.int32),
            pltpu.VMEM((7 * L,), jnp.int32),
            pltpu.VMEM((7 * L,), jnp.int32),
            pltpu.VMEM((ROWS_PER_TILE * H_IMG,), jnp.float32),
        ],
    )(_render_body)
    return f(boxes_flat, mnt_flat)


def kernel(boxes, mount):
    mnt2d = mount[0, 0]
    mnt_flat = jnp.pad(mnt2d, ((0, MNT_STRIDE - MNT), (0, MNT_STRIDE - MNT)))
    mnt_flat = mnt_flat.reshape(MNT_STRIDE * MNT_STRIDE)
    boxes_flat = boxes.reshape(N_BOXES * 4)
    out = _render(boxes_flat, mnt_flat)
    return out.reshape(1, 1, W_IMG, H_IMG)


# speedup vs baseline: 4581.3647x; 1.0011x over previous
import functools

import jax
import jax.numpy as jnp
from jax import lax
from jax.experimental import pallas as pl
from jax.experimental.pallas import tpu as pltpu
from jax.experimental.pallas import tpu_sc as plsc

W_IMG = 512
H_IMG = 512
N_BOXES = 512
MNT = 63
MNT_STRIDE = 64
N_TILES = 32
ROWS_PER_TILE = W_IMG // N_TILES
L = 16


def _render_body(boxes_hbm, mnt_hbm, out_hbm,
                 boxes_v, mnt_v, coords_v, cbuf, canvas, sem):
    cid = lax.axis_index("c")
    sid = lax.axis_index("s")
    t = sid * 2 + cid

    pltpu.sync_copy(boxes_hbm, boxes_v)
    pltpu.sync_copy(mnt_hbm, mnt_v)

    lanes = lax.iota(jnp.int32, L)

    def coord_body(k, carry):
        fb = (k * L + lanes) * 4
        cx = plsc.load_gather(boxes_v, [fb])
        cy = plsc.load_gather(boxes_v, [fb + 1])
        bw = plsc.load_gather(boxes_v, [fb + 2])
        bh = plsc.load_gather(boxes_v, [fb + 3])
        x1 = ((cx - 0.5 * bw) * float(W_IMG)).astype(jnp.int32)
        y1 = ((cy - 0.5 * bh) * float(W_IMG)).astype(jnp.int32)
        x2 = ((cx + 0.5 * bw) * float(W_IMG)).astype(jnp.int32)
        y2 = ((cy + 0.5 * bh) * float(W_IMG)).astype(jnp.int32)
        plsc.store_scatter(coords_v, [fb], x1)
        plsc.store_scatter(coords_v, [fb + 1], y1)
        plsc.store_scatter(coords_v, [fb + 2], x2 - x1)
        plsc.store_scatter(coords_v, [fb + 3], y2 - y1)
        return carry

    lax.fori_loop(0, N_BOXES // L, coord_body, 0)

    def zero_body(k, carry):
        canvas[pl.ds(k * L, L)] = jnp.zeros((L,), jnp.float32)
        return carry

    lax.fori_loop(0, (ROWS_PER_TILE * H_IMG) // L, zero_body, 0)

    def box_body(b, carry):
        cvec = plsc.load_gather(coords_v, [b * 4 + lanes])
        x = cvec[0]
        y = cvec[1]
        w = cvec[2]
        h = cvec[3]
        p0 = x + ((t - x) & (N_TILES - 1))
        n = (x + w - p0 + (N_TILES - 1)) >> 5

        @pl.when(n > 0)
        def _paint():
            wsafe = jnp.maximum(w, 1)
            hsafe = jnp.maximum(h, 1)
            nc = (h + L - 1) // L

            def pre_body(j, c2):
                col = j * L + lanes
                cc = jnp.clip((col * MNT) // hsafe, 0, MNT - 1)
                cbuf[pl.ds(j * L, L)] = jnp.where(col < h, cc, MNT)
                return c2

            lax.fori_loop(0, nc, pre_body, 0)

            rl0 = (p0 - t) >> 5

            def row_body(i, c2):
                px = p0 + i * N_TILES
                rr = jnp.clip(((px - x) * MNT) // wsafe, 0, MNT - 1)
                rbase = rr * MNT_STRIDE
                off = (rl0 + i) * H_IMG + y

                def col_body(j, c3):
                    cc = cbuf[pl.ds(j * L, L)]
                    val = plsc.load_gather(mnt_v, [rbase + cc])
                    s2 = pl.ds(off + j * L, L)
                    canvas[s2] = jnp.maximum(canvas[s2], val)
                    return c3

                lax.fori_loop(0, nc, col_body, 0)
                return c2

            lax.fori_loop(0, n, row_body, 0)

        return carry

    lax.fori_loop(0, N_BOXES, box_body, 0)

    copies = []
    for k in range(ROWS_PER_TILE):
        dst_off = pl.multiple_of((t + N_TILES * k) * H_IMG, 512)
        copies.append(pltpu.async_copy(
            canvas.at[pl.ds(k * H_IMG, H_IMG)],
            out_hbm.at[pl.ds(dst_off, H_IMG)], sem))
    for c in copies:
        c.wait()


@jax.jit
def _render(boxes_flat, mnt_flat):
    mesh = plsc.VectorSubcoreMesh(core_axis_name="c", subcore_axis_name="s")
    f = functools.partial(
        pl.kernel,
        mesh=mesh,
        compiler_params=pltpu.CompilerParams(needs_layout_passes=False),
        out_type=jax.ShapeDtypeStruct((W_IMG * H_IMG,), jnp.float32),
        scratch_types=[
            pltpu.VMEM((N_BOXES * 4,), jnp.float32),
            pltpu.VMEM((MNT_STRIDE * MNT_STRIDE,), jnp.float32),
            pltpu.VMEM((N_BOXES * 4 + L,), jnp.int32),
            pltpu.VMEM((7 * L,), jnp.int32),
            pltpu.VMEM((ROWS_PER_TILE * H_IMG,), jnp.float32),
            pltpu.SemaphoreType.DMA,
        ],
    )(_render_body)
    return f(boxes_flat, mnt_flat)


def kernel(boxes, mount):
    mnt2d = mount[0, 0]
    mnt_flat = jnp.pad(mnt2d, ((0, MNT_STRIDE - MNT), (0, MNT_STRIDE - MNT)))
    mnt_flat = mnt_flat.reshape(MNT_STRIDE * MNT_STRIDE)
    boxes_flat = boxes.reshape(N_BOXES * 4)
    out = _render(boxes_flat, mnt_flat)
    return out.reshape(1, 1, W_IMG, H_IMG)


# speedup vs baseline: 9815.6664x; 2.1425x over previous
import functools

import jax
import jax.numpy as jnp
import numpy as np
from jax import lax
from jax.experimental import pallas as pl
from jax.experimental.pallas import tpu as pltpu
from jax.experimental.pallas import tpu_sc as plsc

W_IMG = 512
H_IMG = 512
N_BOXES = 512
MNT = 63
MNT_STRIDE = 64
N_TILES = 32
ROWS_PER_TILE = W_IMG // N_TILES
L = 16
NREC = 128

_RECIP = np.ones((NREC,), np.float32)
_RECIP[1:] = (1.0 / np.arange(1, NREC)).astype(np.float32)


def _render_body(boxes_hbm, mnt_hbm, recip_hbm, out_hbm,
                 boxes_v, mnt_v, recip_v, coords_v, cbuf, canvas, sem):
    cid = lax.axis_index("c")
    sid = lax.axis_index("s")
    t = sid * 2 + cid

    pltpu.sync_copy(boxes_hbm, boxes_v)
    pltpu.sync_copy(mnt_hbm, mnt_v)
    pltpu.sync_copy(recip_hbm, recip_v)

    lanes = lax.iota(jnp.int32, L)
    zvec = jnp.zeros((L,), jnp.int32)

    def coord_body(k, carry):
        fb = (k * L + lanes) * 4
        cx = plsc.load_gather(boxes_v, [fb])
        cy = plsc.load_gather(boxes_v, [fb + 1])
        bw = plsc.load_gather(boxes_v, [fb + 2])
        bh = plsc.load_gather(boxes_v, [fb + 3])
        x1 = ((cx - 0.5 * bw) * float(W_IMG)).astype(jnp.int32)
        y1 = ((cy - 0.5 * bh) * float(W_IMG)).astype(jnp.int32)
        x2 = ((cx + 0.5 * bw) * float(W_IMG)).astype(jnp.int32)
        y2 = ((cy + 0.5 * bh) * float(W_IMG)).astype(jnp.int32)
        plsc.store_scatter(coords_v, [fb], x1)
        plsc.store_scatter(coords_v, [fb + 1], y1)
        plsc.store_scatter(coords_v, [fb + 2], x2 - x1)
        plsc.store_scatter(coords_v, [fb + 3], y2 - y1)
        return carry

    lax.fori_loop(0, N_BOXES // L, coord_body, 0)

    def zero_body(k, carry):
        canvas[pl.ds(k * L, L)] = jnp.zeros((L,), jnp.float32)
        return carry

    lax.fori_loop(0, (ROWS_PER_TILE * H_IMG) // L, zero_body, 0)

    def box_body(b, carry):
        cvec = plsc.load_gather(coords_v, [b * 4 + lanes])
        x = cvec[0]
        y = cvec[1]
        w = cvec[2]
        h = cvec[3]
        p0 = x + ((t - x) & (N_TILES - 1))
        n = (x + w - p0 + (N_TILES - 1)) >> 5

        @pl.when(n > 0)
        def _paint():
            wsafe = jnp.maximum(w, 1)
            hsafe = jnp.maximum(h, 1)
            nc = (h + L - 1) >> 4
            inv_w = plsc.load_gather(recip_v, [zvec + jnp.minimum(wsafe, NREC - 1)])
            inv_h = plsc.load_gather(recip_v, [zvec + jnp.minimum(hsafe, NREC - 1)])

            def pre_body(j, c2):
                col = j * L + lanes
                cc = (((col * MNT).astype(jnp.float32) + 0.25) * inv_h
                      ).astype(jnp.int32)
                cbuf[pl.ds(j * L, L)] = jnp.where(col < h, cc, MNT)
                return c2

            lax.fori_loop(0, nc, pre_body, 0)

            rl0 = (p0 - t) >> 5

            def row_body(i, c2):
                px = p0 + i * N_TILES
                dx = (zvec + (px - x)) * MNT
                rr = ((dx.astype(jnp.float32) + 0.25) * inv_w).astype(jnp.int32)
                rbase = rr * MNT_STRIDE
                off = (rl0 + i) * H_IMG + y

                def col_body(j, c3):
                    cc = cbuf[pl.ds(j * L, L)]
                    val = plsc.load_gather(mnt_v, [rbase + cc])
                    s2 = pl.ds(off + j * L, L)
                    canvas[s2] = jnp.maximum(canvas[s2], val)
                    return c3

                lax.fori_loop(0, nc, col_body, 0)
                return c2

            lax.fori_loop(0, n, row_body, 0)

        return carry

    lax.fori_loop(0, N_BOXES, box_body, 0)

    copies = []
    for k in range(ROWS_PER_TILE):
        dst_off = pl.multiple_of((t + N_TILES * k) * H_IMG, 512)
        copies.append(pltpu.async_copy(
            canvas.at[pl.ds(k * H_IMG, H_IMG)],
            out_hbm.at[pl.ds(dst_off, H_IMG)], sem))
    for c in copies:
        c.wait()


@jax.jit
def _render(boxes_flat, mnt_flat):
    mesh = plsc.VectorSubcoreMesh(core_axis_name="c", subcore_axis_name="s")
    f = functools.partial(
        pl.kernel,
        mesh=mesh,
        compiler_params=pltpu.CompilerParams(needs_layout_passes=False),
        out_type=jax.ShapeDtypeStruct((W_IMG * H_IMG,), jnp.float32),
        scratch_types=[
            pltpu.VMEM((N_BOXES * 4,), jnp.float32),
            pltpu.VMEM((MNT_STRIDE * MNT_STRIDE,), jnp.float32),
            pltpu.VMEM((NREC,), jnp.float32),
            pltpu.VMEM((N_BOXES * 4 + L,), jnp.int32),
            pltpu.VMEM((7 * L,), jnp.int32),
            pltpu.VMEM((ROWS_PER_TILE * H_IMG,), jnp.float32),
            pltpu.SemaphoreType.DMA,
        ],
    )(_render_body)
    return f(boxes_flat, mnt_flat, jnp.asarray(_RECIP))


def kernel(boxes, mount):
    mnt2d = mount[0, 0]
    mnt_flat = jnp.pad(mnt2d, ((0, MNT_STRIDE - MNT), (0, MNT_STRIDE - MNT)))
    mnt_flat = mnt_flat.reshape(MNT_STRIDE * MNT_STRIDE)
    boxes_flat = boxes.reshape(N_BOXES * 4)
    out = _render(boxes_flat, mnt_flat)
    return out.reshape(1, 1, W_IMG, H_IMG)


# speedup vs baseline: 10046.3365x; 1.0235x over previous
import functools

import jax
import jax.numpy as jnp
import numpy as np
from jax import lax
from jax.experimental import pallas as pl
from jax.experimental.pallas import tpu as pltpu
from jax.experimental.pallas import tpu_sc as plsc

W_IMG = 512
H_IMG = 512
N_BOXES = 512
MNT = 63
MNT_STRIDE = 64
N_TILES = 32
ROWS_PER_TILE = W_IMG // N_TILES
L = 16
TDIM = 128

_A = np.arange(TDIM, dtype=np.int64)
_D = np.maximum(_A, 1)[:, None]
_CTABLE = np.where(_A[None, :] < _D, (_A[None, :] * MNT) // _D, MNT)
_CTABLE = _CTABLE.astype(np.int32).reshape(TDIM * TDIM)


def _render_body(boxes_hbm, mnt_hbm, ctab_hbm, out_hbm,
                 boxes_v, mnt_v, ctab_v, coords_v, canvas, sem):
    cid = lax.axis_index("c")
    sid = lax.axis_index("s")
    t = sid * 2 + cid

    pltpu.sync_copy(boxes_hbm, boxes_v)
    pltpu.sync_copy(mnt_hbm, mnt_v)
    pltpu.sync_copy(ctab_hbm, ctab_v)

    lanes = lax.iota(jnp.int32, L)
    zvec = jnp.zeros((L,), jnp.int32)

    def coord_body(k, carry):
        fb = (k * L + lanes) * 4
        cx = plsc.load_gather(boxes_v, [fb])
        cy = plsc.load_gather(boxes_v, [fb + 1])
        bw = plsc.load_gather(boxes_v, [fb + 2])
        bh = plsc.load_gather(boxes_v, [fb + 3])
        x1 = ((cx - 0.5 * bw) * float(W_IMG)).astype(jnp.int32)
        y1 = ((cy - 0.5 * bh) * float(W_IMG)).astype(jnp.int32)
        x2 = ((cx + 0.5 * bw) * float(W_IMG)).astype(jnp.int32)
        y2 = ((cy + 0.5 * bh) * float(W_IMG)).astype(jnp.int32)
        plsc.store_scatter(coords_v, [fb], x1)
        plsc.store_scatter(coords_v, [fb + 1], y1)
        plsc.store_scatter(coords_v, [fb + 2], x2 - x1)
        plsc.store_scatter(coords_v, [fb + 3], y2 - y1)
        return carry

    lax.fori_loop(0, N_BOXES // L, coord_body, 0)

    def zero_body(k, carry):
        canvas[pl.ds(k * L, L)] = jnp.zeros((L,), jnp.float32)
        return carry

    lax.fori_loop(0, (ROWS_PER_TILE * H_IMG) // L, zero_body, 0)

    def box_body(b, carry):
        cvec = plsc.load_gather(coords_v, [b * 4 + lanes])
        x = cvec[0]
        y = cvec[1]
        w = cvec[2]
        h = cvec[3]
        p0 = x + ((t - x) & (N_TILES - 1))
        n = (x + w - p0 + (N_TILES - 1)) >> 5

        @pl.when(n > 0)
        def _paint():
            wbase = jnp.minimum(jnp.maximum(w, 1), TDIM - 1) << 7
            hbase = jnp.minimum(jnp.maximum(h, 1), TDIM - 1) << 7
            nc = (h + L - 1) >> 4
            rl0 = (p0 - t) >> 5

            def row_body(i, c2):
                px = p0 + i * N_TILES
                rr = plsc.load_gather(ctab_v, [zvec + (wbase + px - x)])
                rbase = rr * MNT_STRIDE
                off = (rl0 + i) * H_IMG + y

                def col_body(j, c3):
                    cc = ctab_v[pl.ds(hbase + j * L, L)]
                    val = plsc.load_gather(mnt_v, [rbase + cc])
                    s2 = pl.ds(off + j * L, L)
                    canvas[s2] = jnp.maximum(canvas[s2], val)
                    return c3

                lax.fori_loop(0, nc, col_body, 0)
                return c2

            lax.fori_loop(0, n, row_body, 0)

        return carry

    lax.fori_loop(0, N_BOXES, box_body, 0)

    copies = []
    for k in range(ROWS_PER_TILE):
        dst_off = pl.multiple_of((t + N_TILES * k) * H_IMG, 512)
        copies.append(pltpu.async_copy(
            canvas.at[pl.ds(k * H_IMG, H_IMG)],
            out_hbm.at[pl.ds(dst_off, H_IMG)], sem))
    for c in copies:
        c.wait()


@jax.jit
def _render(boxes_flat, mnt_flat):
    mesh = plsc.VectorSubcoreMesh(core_axis_name="c", subcore_axis_name="s")
    f = functools.partial(
        pl.kernel,
        mesh=mesh,
        compiler_params=pltpu.CompilerParams(needs_layout_passes=False),
        out_type=jax.ShapeDtypeStruct((W_IMG * H_IMG,), jnp.float32),
        scratch_types=[
            pltpu.VMEM((N_BOXES * 4,), jnp.float32),
            pltpu.VMEM((MNT_STRIDE * MNT_STRIDE,), jnp.float32),
            pltpu.VMEM((TDIM * TDIM,), jnp.int32),
            pltpu.VMEM((N_BOXES * 4 + L,), jnp.int32),
            pltpu.VMEM((ROWS_PER_TILE * H_IMG,), jnp.float32),
            pltpu.SemaphoreType.DMA,
        ],
    )(_render_body)
    return f(boxes_flat, mnt_flat, jnp.asarray(_CTABLE))


def kernel(boxes, mount):
    mnt2d = mount[0, 0]
    mnt_flat = jnp.pad(mnt2d, ((0, MNT_STRIDE - MNT), (0, MNT_STRIDE - MNT)))
    mnt_flat = mnt_flat.reshape(MNT_STRIDE * MNT_STRIDE)
    boxes_flat = boxes.reshape(N_BOXES * 4)
    out = _render(boxes_flat, mnt_flat)
    return out.reshape(1, 1, W_IMG, H_IMG)


# speedup vs baseline: 10455.5006x; 1.0407x over previous
import functools

import jax
import jax.numpy as jnp
import numpy as np
from jax import lax
from jax.experimental import pallas as pl
from jax.experimental.pallas import tpu as pltpu
from jax.experimental.pallas import tpu_sc as plsc

W_IMG = 512
H_IMG = 512
N_BOXES = 512
MNT = 63
MNT_STRIDE = 64
N_TILES = 32
ROWS_PER_TILE = W_IMG // N_TILES
L = 16
TDIM = 128

_A = np.arange(TDIM, dtype=np.int64)
_D = np.maximum(_A, 1)[:, None]
_CTABLE = np.where(_A[None, :] < _D, (_A[None, :] * MNT) // _D, MNT)
_CTABLE = _CTABLE.astype(np.int32).reshape(TDIM * TDIM)


def _render_body(boxes_hbm, mnt_hbm, ctab_hbm, out_hbm,
                 boxes_v, mnt_v, ctab_v, coords_v, canvas, sem):
    cid = lax.axis_index("c")
    sid = lax.axis_index("s")
    t = sid * 2 + cid

    pltpu.sync_copy(boxes_hbm, boxes_v)
    pltpu.sync_copy(mnt_hbm, mnt_v)
    pltpu.sync_copy(ctab_hbm, ctab_v)

    lanes = lax.iota(jnp.int32, L)
    zvec = jnp.zeros((L,), jnp.int32)

    def coord_body(k, carry):
        fb = (k * L + lanes) * 4
        cx = plsc.load_gather(boxes_v, [fb])
        cy = plsc.load_gather(boxes_v, [fb + 1])
        bw = plsc.load_gather(boxes_v, [fb + 2])
        bh = plsc.load_gather(boxes_v, [fb + 3])
        x1 = ((cx - 0.5 * bw) * float(W_IMG)).astype(jnp.int32)
        y1 = ((cy - 0.5 * bh) * float(W_IMG)).astype(jnp.int32)
        x2 = ((cx + 0.5 * bw) * float(W_IMG)).astype(jnp.int32)
        y2 = ((cy + 0.5 * bh) * float(W_IMG)).astype(jnp.int32)
        plsc.store_scatter(coords_v, [fb], x1)
        plsc.store_scatter(coords_v, [fb + 1], y1)
        plsc.store_scatter(coords_v, [fb + 2], x2 - x1)
        plsc.store_scatter(coords_v, [fb + 3], y2 - y1)
        return carry

    lax.fori_loop(0, N_BOXES // L, coord_body, 0)

    def zero_body(k, carry):
        for u in range(4):
            canvas[pl.ds((k * 4 + u) * L, L)] = jnp.zeros((L,), jnp.float32)
        return carry

    lax.fori_loop(0, ((ROWS_PER_TILE + 1) * H_IMG) // (4 * L), zero_body, 0)

    def box_body(b, carry):
        cvec = plsc.load_gather(coords_v, [b * 4 + lanes])
        x = cvec[0]
        y = cvec[1]
        w = cvec[2]
        h = cvec[3]
        p0 = x + ((t - x) & (N_TILES - 1))
        wbase = jnp.minimum(jnp.maximum(w, 1), TDIM - 1) << 7
        hbase = jnp.minimum(jnp.maximum(h, 1), TDIM - 1) << 7
        rl0 = (p0 - t) >> 5
        ridx = jnp.minimum(wbase + (p0 - x) + lanes * N_TILES,
                           wbase + TDIM - 1)
        rvec = plsc.load_gather(ctab_v, [ridx]) * MNT_STRIDE
        ccs = [ctab_v[pl.ds(hbase + j * L, L)] for j in range(7)]
        for i in range(4):
            rb = rvec[i]
            off = jnp.minimum(rl0 + i, ROWS_PER_TILE) * H_IMG + y
            for j in range(7):
                val = plsc.load_gather(mnt_v, [rb + ccs[j]])
                s2 = pl.ds(off + j * L, L)
                canvas[s2] = jnp.maximum(canvas[s2], val)
        return carry

    lax.fori_loop(0, N_BOXES, box_body, 0)

    copies = []
    for k in range(ROWS_PER_TILE):
        dst_off = pl.multiple_of((t + N_TILES * k) * H_IMG, 512)
        copies.append(pltpu.async_copy(
            canvas.at[pl.ds(k * H_IMG, H_IMG)],
            out_hbm.at[pl.ds(dst_off, H_IMG)], sem))
    for c in copies:
        c.wait()


@jax.jit
def _render(boxes_flat, mnt_flat):
    mesh = plsc.VectorSubcoreMesh(core_axis_name="c", subcore_axis_name="s")
    f = functools.partial(
        pl.kernel,
        mesh=mesh,
        compiler_params=pltpu.CompilerParams(needs_layout_passes=False),
        out_type=jax.ShapeDtypeStruct((W_IMG * H_IMG,), jnp.float32),
        scratch_types=[
            pltpu.VMEM((N_BOXES * 4,), jnp.float32),
            pltpu.VMEM((MNT_STRIDE * MNT_STRIDE,), jnp.float32),
            pltpu.VMEM((TDIM * TDIM,), jnp.int32),
            pltpu.VMEM((N_BOXES * 4 + L,), jnp.int32),
            pltpu.VMEM(((ROWS_PER_TILE + 1) * H_IMG,), jnp.float32),
            pltpu.SemaphoreType.DMA,
        ],
    )(_render_body)
    return f(boxes_flat, mnt_flat, jnp.asarray(_CTABLE))


def kernel(boxes, mount):
    mnt2d = mount[0, 0]
    mnt_flat = jnp.pad(mnt2d, ((0, MNT_STRIDE - MNT), (0, MNT_STRIDE - MNT)))
    mnt_flat = mnt_flat.reshape(MNT_STRIDE * MNT_STRIDE)
    boxes_flat = boxes.reshape(N_BOXES * 4)
    out = _render(boxes_flat, mnt_flat)
    return out.reshape(1, 1, W_IMG, H_IMG)


# speedup vs baseline: 12454.2408x; 1.1912x over previous
import functools

import jax
import jax.numpy as jnp
import numpy as np
from jax import lax
from jax.experimental import pallas as pl
from jax.experimental.pallas import tpu as pltpu
from jax.experimental.pallas import tpu_sc as plsc

W_IMG = 512
H_IMG = 512
N_BOXES = 512
MNT = 63
MNT_STRIDE = 64
N_TILES = 32
ROWS_PER_TILE = W_IMG // N_TILES
L = 16
TDIM = 128

_A = np.arange(TDIM, dtype=np.int64)
_D = np.maximum(_A, 1)[:, None]
_CTABLE = np.where(_A[None, :] < _D, (_A[None, :] * MNT) // _D, MNT)
_CTABLE = _CTABLE.astype(np.int32).reshape(TDIM * TDIM)


def _render_body(boxes_hbm, mnt_hbm, ctab_hbm, out_hbm,
                 boxes_v, mnt_v, ctab_v, coords_v, rbuf_v, canvas, sem):
    cid = lax.axis_index("c")
    sid = lax.axis_index("s")
    t = sid * 2 + cid

    pltpu.sync_copy(boxes_hbm, boxes_v)
    pltpu.sync_copy(mnt_hbm, mnt_v)
    pltpu.sync_copy(ctab_hbm, ctab_v)

    lanes = lax.iota(jnp.int32, L)
    zvec = jnp.zeros((L,), jnp.int32)

    def coord_body(k, carry):
        fb = (k * L + lanes) * 4
        cx = plsc.load_gather(boxes_v, [fb])
        cy = plsc.load_gather(boxes_v, [fb + 1])
        bw = plsc.load_gather(boxes_v, [fb + 2])
        bh = plsc.load_gather(boxes_v, [fb + 3])
        x1 = ((cx - 0.5 * bw) * float(W_IMG)).astype(jnp.int32)
        y1 = ((cy - 0.5 * bh) * float(W_IMG)).astype(jnp.int32)
        x2 = ((cx + 0.5 * bw) * float(W_IMG)).astype(jnp.int32)
        y2 = ((cy + 0.5 * bh) * float(W_IMG)).astype(jnp.int32)
        plsc.store_scatter(coords_v, [fb], x1)
        plsc.store_scatter(coords_v, [fb + 1], y1)
        plsc.store_scatter(coords_v, [fb + 2], x2 - x1)
        plsc.store_scatter(coords_v, [fb + 3], y2 - y1)
        return carry

    lax.fori_loop(0, N_BOXES // L, coord_body, 0)

    def zero_body(k, carry):
        for u in range(4):
            canvas[pl.ds((k * 4 + u) * L, L)] = jnp.zeros((L,), jnp.float32)
        return carry

    lax.fori_loop(0, ((ROWS_PER_TILE + 1) * H_IMG) // (4 * L), zero_body, 0)

    def box_body(b, carry):
        cvec = plsc.load_gather(coords_v, [b * 4 + lanes])
        x = cvec[0]
        y = cvec[1]
        w = cvec[2]
        h = cvec[3]
        p0 = x + ((t - x) & (N_TILES - 1))
        n = (x + w - p0 + (N_TILES - 1)) >> 5
        wbase = jnp.minimum(jnp.maximum(w, 1), TDIM - 1) << 7
        hbase = jnp.minimum(jnp.maximum(h, 1), TDIM - 1) << 7
        rl0 = (p0 - t) >> 5
        nc2 = (h + 2 * L - 1) >> 5
        ridx = jnp.minimum(wbase + (p0 - x) + lanes * N_TILES,
                           wbase + TDIM - 1)
        rbuf_v[pl.ds(0, L)] = plsc.load_gather(ctab_v, [ridx]) * MNT_STRIDE

        def row_body(i, c2):
            rb = plsc.load_gather(rbuf_v, [zvec + i])
            off = (rl0 + i) * H_IMG + y

            def col_body(jj, c3):
                cc0 = ctab_v[pl.ds(hbase + jj * (2 * L), L)]
                cc1 = ctab_v[pl.ds(hbase + jj * (2 * L) + L, L)]
                val0 = plsc.load_gather(mnt_v, [rb + cc0])
                val1 = plsc.load_gather(mnt_v, [rb + cc1])
                s0 = pl.ds(off + jj * (2 * L), L)
                s1 = pl.ds(off + jj * (2 * L) + L, L)
                canvas[s0] = jnp.maximum(canvas[s0], val0)
                canvas[s1] = jnp.maximum(canvas[s1], val1)
                return c3

            lax.fori_loop(0, nc2, col_body, 0)
            return c2

        lax.fori_loop(0, n, row_body, 0)
        return carry

    lax.fori_loop(0, N_BOXES, box_body, 0)

    copies = []
    for k in range(ROWS_PER_TILE):
        dst_off = pl.multiple_of((t + N_TILES * k) * H_IMG, 512)
        copies.append(pltpu.async_copy(
            canvas.at[pl.ds(k * H_IMG, H_IMG)],
            out_hbm.at[pl.ds(dst_off, H_IMG)], sem))
    for c in copies:
        c.wait()


@jax.jit
def _render(boxes_flat, mnt_flat):
    mesh = plsc.VectorSubcoreMesh(core_axis_name="c", subcore_axis_name="s")
    f = functools.partial(
        pl.kernel,
        mesh=mesh,
        compiler_params=pltpu.CompilerParams(needs_layout_passes=False),
        out_type=jax.ShapeDtypeStruct((W_IMG * H_IMG,), jnp.float32),
        scratch_types=[
            pltpu.VMEM((N_BOXES * 4,), jnp.float32),
            pltpu.VMEM((MNT_STRIDE * MNT_STRIDE,), jnp.float32),
            pltpu.VMEM((TDIM * TDIM,), jnp.int32),
            pltpu.VMEM((N_BOXES * 4 + L,), jnp.int32),
            pltpu.VMEM((L,), jnp.int32),
            pltpu.VMEM(((ROWS_PER_TILE + 1) * H_IMG,), jnp.float32),
            pltpu.SemaphoreType.DMA,
        ],
    )(_render_body)
    return f(boxes_flat, mnt_flat, jnp.asarray(_CTABLE))


def kernel(boxes, mount):
    mnt2d = mount[0, 0]
    mnt_flat = jnp.pad(mnt2d, ((0, MNT_STRIDE - MNT), (0, MNT_STRIDE - MNT)))
    mnt_flat = mnt_flat.reshape(MNT_STRIDE * MNT_STRIDE)
    boxes_flat = boxes.reshape(N_BOXES * 4)
    out = _render(boxes_flat, mnt_flat)
    return out.reshape(1, 1, W_IMG, H_IMG)


# speedup vs baseline: 15889.2763x; 1.2758x over previous
import functools

import jax
import jax.numpy as jnp
import numpy as np
from jax import lax
from jax.experimental import pallas as pl
from jax.experimental.pallas import tpu as pltpu
from jax.experimental.pallas import tpu_sc as plsc

W_IMG = 512
H_IMG = 512
N_BOXES = 512
MNT = 63
MNT_STRIDE = 64
N_TILES = 32
ROWS_PER_TILE = W_IMG // N_TILES
L = 16
TDIM = 128
NF = 16

_A = np.arange(TDIM, dtype=np.int64)
_D = np.maximum(_A, 1)[:, None]
_CTABLE = np.where(_A[None, :] < _D, (_A[None, :] * MNT) // _D, MNT)
_CTABLE = _CTABLE.astype(np.int32).reshape(TDIM * TDIM)


def _render_body(boxes_hbm, mnt_hbm, ctab_hbm, out_hbm,
                 boxes_v, mnt_v, ctab_v, fields_v, canvas, sem):
    cid = lax.axis_index("c")
    sid = lax.axis_index("s")
    t = sid * 2 + cid

    pltpu.sync_copy(boxes_hbm, boxes_v)
    pltpu.sync_copy(mnt_hbm, mnt_v)
    pltpu.sync_copy(ctab_hbm, ctab_v)

    lanes = lax.iota(jnp.int32, L)
    lanes32 = lanes * N_TILES

    def coord_body(k, carry):
        fb = (k * L + lanes) * 4
        cx = plsc.load_gather(boxes_v, [fb])
        cy = plsc.load_gather(boxes_v, [fb + 1])
        bw = plsc.load_gather(boxes_v, [fb + 2])
        bh = plsc.load_gather(boxes_v, [fb + 3])
        x1 = ((cx - 0.5 * bw) * float(W_IMG)).astype(jnp.int32)
        y1 = ((cy - 0.5 * bh) * float(W_IMG)).astype(jnp.int32)
        x2 = ((cx + 0.5 * bw) * float(W_IMG)).astype(jnp.int32)
        y2 = ((cy + 0.5 * bh) * float(W_IMG)).astype(jnp.int32)
        w = x2 - x1
        h = y2 - y1
        wb = jnp.minimum(jnp.maximum(w, 1), TDIM - 1) << 7
        hb = jnp.minimum(jnp.maximum(h, 1), TDIM - 1) << 7
        dx0 = (t - x1) & (N_TILES - 1)
        rl0 = (x1 + dx0 - t) >> 5
        nc2 = (h + 2 * L - 1) >> 5
        off0 = (rl0 << 9) + y1
        off1 = (jnp.minimum(rl0 + 1, ROWS_PER_TILE) << 9) + y1
        off2 = (jnp.minimum(rl0 + 2, ROWS_PER_TILE) << 9) + y1
        off3 = (jnp.minimum(rl0 + 3, ROWS_PER_TILE) << 9) + y1
        fo = (k * L + lanes) * NF
        plsc.store_scatter(fields_v, [fo], wb + dx0)
        plsc.store_scatter(fields_v, [fo + 1], wb + TDIM - 1)
        plsc.store_scatter(fields_v, [fo + 2], hb)
        plsc.store_scatter(fields_v, [fo + 3], nc2)
        plsc.store_scatter(fields_v, [fo + 4], off0)
        plsc.store_scatter(fields_v, [fo + 5], off1)
        plsc.store_scatter(fields_v, [fo + 6], off2)
        plsc.store_scatter(fields_v, [fo + 7], off3)
        plsc.store_scatter(fields_v, [fo + 8], w - dx0)
        return carry

    lax.fori_loop(0, N_BOXES // L, coord_body, 0)

    def zero_body(k, carry):
        for u in range(4):
            canvas[pl.ds((k * 4 + u) * L, L)] = jnp.zeros((L,), jnp.float32)
        return carry

    lax.fori_loop(0, ((ROWS_PER_TILE + 1) * H_IMG) // (4 * L), zero_body, 0)

    def box_body(b, fvec):
        nvec = plsc.load_gather(fields_v, [(b + 1) * NF + lanes])
        wbdx = fvec[0]
        wb127 = fvec[1]
        hb = fvec[2]
        nc2 = fvec[3]
        off0 = fvec[4]
        off1 = fvec[5]
        off2 = fvec[6]
        off3 = fvec[7]
        nfl = fvec[8]
        ridx = jnp.minimum(wbdx + lanes32, wb127)
        rvec = plsc.load_gather(ctab_v, [ridx]) << 6
        rb0 = rvec[0]
        rb1 = rvec[1]

        def col01(jj, c2):
            base = hb + jj * (2 * L)
            cc0 = ctab_v[pl.ds(base, L)]
            cc1 = ctab_v[pl.ds(base + L, L)]
            for rb, off in ((rb0, off0), (rb1, off1)):
                v0 = plsc.load_gather(mnt_v, [rb + cc0])
                v1 = plsc.load_gather(mnt_v, [rb + cc1])
                s0 = pl.ds(off + jj * (2 * L), L)
                s1 = pl.ds(off + jj * (2 * L) + L, L)
                canvas[s0] = jnp.maximum(canvas[s0], v0)
                canvas[s1] = jnp.maximum(canvas[s1], v1)
            return c2

        lax.fori_loop(0, nc2, col01, 0)

        @pl.when(nfl > 2 * N_TILES)
        def _rows23():
            rb2 = rvec[2]
            rb3 = rvec[3]

            def col23(jj, c2):
                base = hb + jj * (2 * L)
                cc0 = ctab_v[pl.ds(base, L)]
                cc1 = ctab_v[pl.ds(base + L, L)]
                for rb, off in ((rb2, off2), (rb3, off3)):
                    v0 = plsc.load_gather(mnt_v, [rb + cc0])
                    v1 = plsc.load_gather(mnt_v, [rb + cc1])
                    s0 = pl.ds(off + jj * (2 * L), L)
                    s1 = pl.ds(off + jj * (2 * L) + L, L)
                    canvas[s0] = jnp.maximum(canvas[s0], v0)
                    canvas[s1] = jnp.maximum(canvas[s1], v1)
                return c2

            lax.fori_loop(0, nc2, col23, 0)

        return nvec

    fvec0 = plsc.load_gather(fields_v, [lanes])
    lax.fori_loop(0, N_BOXES, box_body, fvec0)

    copies = []
    for k in range(ROWS_PER_TILE):
        dst_off = pl.multiple_of((t + N_TILES * k) * H_IMG, 512)
        copies.append(pltpu.async_copy(
            canvas.at[pl.ds(k * H_IMG, H_IMG)],
            out_hbm.at[pl.ds(dst_off, H_IMG)], sem))
    for c in copies:
        c.wait()


@jax.jit
def _render(boxes_flat, mnt_flat):
    mesh = plsc.VectorSubcoreMesh(core_axis_name="c", subcore_axis_name="s")
    f = functools.partial(
        pl.kernel,
        mesh=mesh,
        compiler_params=pltpu.CompilerParams(needs_layout_passes=False),
        out_type=jax.ShapeDtypeStruct((W_IMG * H_IMG,), jnp.float32),
        scratch_types=[
            pltpu.VMEM((N_BOXES * 4,), jnp.float32),
            pltpu.VMEM((MNT_STRIDE * MNT_STRIDE,), jnp.float32),
            pltpu.VMEM((TDIM * TDIM,), jnp.int32),
            pltpu.VMEM(((N_BOXES + 1) * NF + L,), jnp.int32),
            pltpu.VMEM(((ROWS_PER_TILE + 1) * H_IMG,), jnp.float32),
            pltpu.SemaphoreType.DMA,
        ],
    )(_render_body)
    return f(boxes_flat, mnt_flat, jnp.asarray(_CTABLE))


def kernel(boxes, mount):
    mnt2d = mount[0, 0]
    mnt_flat = jnp.pad(mnt2d, ((0, MNT_STRIDE - MNT), (0, MNT_STRIDE - MNT)))
    mnt_flat = mnt_flat.reshape(MNT_STRIDE * MNT_STRIDE)
    boxes_flat = boxes.reshape(N_BOXES * 4)
    out = _render(boxes_flat, mnt_flat)
    return out.reshape(1, 1, W_IMG, H_IMG)
